# baseline (speedup 1.0000x reference)
import functools

import jax
import jax.numpy as jnp
from jax import lax
from jax.experimental import pallas as pl
from jax.experimental.pallas import tpu as pltpu
from jax.experimental.pallas import tpu_sc as plsc

_B = 16384
_L = 20
_D = 300
_V = 100000
_DP = 384
_NPART = 3
_P2OFF = 256
_NC = 2
_NS = 16
_NW = _NC * _NS
_BPW = _B // _NW
_CB = 4
_G = _CB * _L
_NCHUNK = _BPW // _CB
_COLS = tuple(range(0, _D - 16, 16)) + (_D - 16,)
_VBLK = 1000


def _plane_off(s):
    if s < 128:
        return 0, s
    if s + 16 <= 256:
        return 1, s - 128
    return 2, s - _P2OFF


def _relayout_body(t2_ref, p2_ref):
    p2_ref[...] = t2_ref[...]


def _relayout(table):
    return pl.pallas_call(
        _relayout_body,
        grid=(_V // _VBLK,),
        in_specs=[pl.BlockSpec((_VBLK, 128), lambda i: (i, 2))],
        out_specs=pl.BlockSpec((_VBLK, 128), lambda i: (i, 0)),
        out_shape=jax.ShapeDtypeStruct((_V, 128), jnp.float32),
    )(table)


def _accumulate_chunk(rows_v, out_v, obase):
    def body(i, carry):
        accs = [jnp.zeros((16,), jnp.float32) for _ in _COLS]
        cnt_v = jnp.zeros((16,), jnp.float32)
        zero_v = jnp.zeros((16,), jnp.float32)
        one_v = jnp.ones((16,), jnp.float32)
        r0 = i * _L
        for l in range(_L):
            rowm = jnp.zeros((16,), jnp.float32)
            for c, s in enumerate(_COLS):
                p, off = _plane_off(s)
                v = rows_v[p * _G + r0 + l, pl.ds(off, 16)]
                accs[c] = accs[c] + v
                rowm = jnp.maximum(rowm, jnp.abs(v))
            pop = plsc.all_reduce_population_count(rowm > 0.0)
            cnt_v = cnt_v + jnp.where(pop > 0, one_v, zero_v)
        inv_v = jnp.where(cnt_v > 0.0, one_v / jnp.maximum(cnt_v, one_v), zero_v)
        for c, s in enumerate(_COLS):
            out_v[obase + _NPART * i + s // 128, pl.ds(s % 128, 16)] = (
                accs[c] * inv_v)
        return carry

    lax.fori_loop(0, _CB, body, 0)


def _make_kernel():
    mesh = plsc.VectorSubcoreMesh(core_axis_name="c", subcore_axis_name="s")

    @functools.partial(
        pl.kernel,
        mesh=mesh,
        out_type=jax.ShapeDtypeStruct((_NPART * _B, 128), jnp.float32),
        compiler_params=pltpu.CompilerParams(
            needs_layout_passes=False, use_tc_tiling_on_sc=True),
        scratch_types=[
            pltpu.VMEM((_G,), jnp.int32),
            pltpu.VMEM((_G,), jnp.int32),
            pltpu.VMEM((_NPART * _G, 128), jnp.float32),
            pltpu.VMEM((_NPART * _G, 128), jnp.float32),
            pltpu.VMEM((2 * _NPART * _CB, 128), jnp.float32),
            pltpu.SemaphoreType.DMA,
            pltpu.SemaphoreType.DMA,
            pltpu.SemaphoreType.DMA,
            pltpu.SemaphoreType.DMA,
        ],
    )
    def category_kernel(idx_hbm, table_hbm, p2_hbm, out_hbm,
                        raw0, raw1, rows0, rows1, out_v,
                        sem_i0, sem_i1, sem_g0, sem_g1):
        wid = lax.axis_index("s") * _NC + lax.axis_index("c")
        wbase = wid * _BPW

        raw_bufs = (raw0, raw1)
        row_bufs = (rows0, rows1)
        sem_i = (sem_i0, sem_i1)
        sem_g = (sem_g0, sem_g1)

        def raw_copy(chunk, buf):
            return pltpu.make_async_copy(
                idx_hbm.at[pl.ds((wbase + chunk * _CB) * _L, _G)],
                raw_bufs[buf], sem_i[buf])

        def gather_copies(buf):
            copies = [
                pltpu.make_async_copy(
                    table_hbm.at[raw_bufs[buf], pl.ds(128 * p, 128)],
                    row_bufs[buf].at[pl.ds(p * _G, _G)],
                    sem_g[buf])
                for p in range(2)
            ]
            copies.append(pltpu.make_async_copy(
                p2_hbm.at[raw_bufs[buf]],
                row_bufs[buf].at[pl.ds(2 * _G, _G)],
                sem_g[buf]))
            return copies

        def start_gathers(buf):
            for cp in gather_copies(buf):
                cp.start()

        def wait_gathers(buf):
            for cp in gather_copies(buf):
                cp.wait()

        def process(chunk, buf):
            _accumulate_chunk(row_bufs[buf], out_v, buf * _NPART * _CB)
            if buf == 1:
                pltpu.sync_copy(
                    out_v,
                    out_hbm.at[pl.ds(_NPART * (wbase + (chunk - 1) * _CB),
                                     2 * _NPART * _CB)])

        raw_copy(0, 0).start()
        raw_copy(1, 1).start()
        raw_copy(0, 0).wait()
        start_gathers(0)

        def half_step(g, buf):
            nxt = 1 - buf
            wait_gathers(buf)

            @pl.when(g + 1 < _NCHUNK)
            def _():
                raw_copy(g + 1, nxt).wait()
                start_gathers(nxt)

            @pl.when(g + 2 < _NCHUNK)
            def _():
                raw_copy(g + 2, buf).start()

            process(g, buf)

        def outer(i, carry):
            g = i * 2
            half_step(g, 0)
            half_step(g + 1, 1)
            return carry

        lax.fori_loop(0, _NCHUNK // 2, outer, 0)

    return category_kernel


_kernel_call = _make_kernel()


def kernel(indices, table):
    idx_flat = indices.astype(jnp.int32).reshape(_B * _L)
    p2 = _relayout(table)
    out128 = _kernel_call(idx_flat, table, p2)
    return out128.reshape(_B, _DP)[:, :_D]


# speedup vs baseline: 1.1223x; 1.1223x over previous
import functools

import jax
import jax.numpy as jnp
from jax import lax
from jax.experimental import pallas as pl
from jax.experimental.pallas import tpu as pltpu
from jax.experimental.pallas import tpu_sc as plsc

_B = 16384
_L = 20
_D = 300
_V = 100000
_DP = 384
_NPART = 3
_P2OFF = 256
_NC = 2
_NS = 16
_NW = _NC * _NS
_BPW = _B // _NW
_CB = 4
_G = _CB * _L
_NCHUNK = _BPW // _CB
_COLS = tuple(range(0, _D - 16, 16)) + (_D - 16,)
_VBLK = 10000


def _plane_off(s):
    if s < 128:
        return 0, s
    if s + 16 <= 256:
        return 1, s - 128
    return 2, s - _P2OFF


def _relayout_body(t2_ref, p2_ref):
    p2_ref[...] = t2_ref[...]


def _relayout(table):
    return pl.pallas_call(
        _relayout_body,
        grid=(_V // _VBLK,),
        in_specs=[pl.BlockSpec((_VBLK, 128), lambda i: (i, 2))],
        out_specs=pl.BlockSpec((_VBLK, 128), lambda i: (i, 0)),
        out_shape=jax.ShapeDtypeStruct((_V, 128), jnp.float32),
    )(table)


def _accumulate_chunk(rows_v, out_v, obase):
    def body(i, carry):
        accs = [jnp.zeros((16,), jnp.float32) for _ in _COLS]
        cnt_v = jnp.zeros((16,), jnp.float32)
        zero_v = jnp.zeros((16,), jnp.float32)
        one_v = jnp.ones((16,), jnp.float32)
        r0 = i * _L
        for l in range(_L):
            rowm = jnp.zeros((16,), jnp.float32)
            for c, s in enumerate(_COLS):
                p, off = _plane_off(s)
                v = rows_v[p * _G + r0 + l, pl.ds(off, 16)]
                accs[c] = accs[c] + v
                rowm = jnp.maximum(rowm, jnp.abs(v))
            pop = plsc.all_reduce_population_count(rowm > 0.0)
            cnt_v = cnt_v + jnp.where(pop > 0, one_v, zero_v)
        inv_v = jnp.where(cnt_v > 0.0, one_v / jnp.maximum(cnt_v, one_v), zero_v)
        for c, s in enumerate(_COLS):
            out_v[obase + i, pl.ds(s, 16)] = accs[c] * inv_v
        return carry

    lax.fori_loop(0, _CB, body, 0)


def _make_kernel():
    mesh = plsc.VectorSubcoreMesh(core_axis_name="c", subcore_axis_name="s")

    @functools.partial(
        pl.kernel,
        mesh=mesh,
        out_type=jax.ShapeDtypeStruct((_B, _D), jnp.float32),
        compiler_params=pltpu.CompilerParams(
            needs_layout_passes=False, use_tc_tiling_on_sc=True),
        scratch_types=[
            pltpu.VMEM((_G,), jnp.int32),
            pltpu.VMEM((_G,), jnp.int32),
            pltpu.VMEM((_NPART * _G, 128), jnp.float32),
            pltpu.VMEM((_NPART * _G, 128), jnp.float32),
            pltpu.VMEM((2 * _CB, _D), jnp.float32),
            pltpu.SemaphoreType.DMA,
            pltpu.SemaphoreType.DMA,
            pltpu.SemaphoreType.DMA,
            pltpu.SemaphoreType.DMA,
        ],
    )
    def category_kernel(idx_hbm, table_hbm, p2_hbm, out_hbm,
                        raw0, raw1, rows0, rows1, out_v,
                        sem_i0, sem_i1, sem_g0, sem_g1):
        wid = lax.axis_index("s") * _NC + lax.axis_index("c")
        wbase = wid * _BPW

        raw_bufs = (raw0, raw1)
        row_bufs = (rows0, rows1)
        sem_i = (sem_i0, sem_i1)
        sem_g = (sem_g0, sem_g1)

        def raw_copy(chunk, buf):
            return pltpu.make_async_copy(
                idx_hbm.at[pl.ds((wbase + chunk * _CB) * _L, _G)],
                raw_bufs[buf], sem_i[buf])

        def gather_copies(buf):
            copies = [
                pltpu.make_async_copy(
                    table_hbm.at[raw_bufs[buf], pl.ds(128 * p, 128)],
                    row_bufs[buf].at[pl.ds(p * _G, _G)],
                    sem_g[buf])
                for p in range(2)
            ]
            copies.append(pltpu.make_async_copy(
                p2_hbm.at[raw_bufs[buf]],
                row_bufs[buf].at[pl.ds(2 * _G, _G)],
                sem_g[buf]))
            return copies

        def start_gathers(buf):
            for cp in gather_copies(buf):
                cp.start()

        def wait_gathers(buf):
            for cp in gather_copies(buf):
                cp.wait()

        def process(chunk, buf):
            _accumulate_chunk(row_bufs[buf], out_v, buf * _CB)
            if buf == 1:
                pltpu.sync_copy(
                    out_v,
                    out_hbm.at[pl.ds(wbase + (chunk - 1) * _CB, 2 * _CB)])

        raw_copy(0, 0).start()
        raw_copy(1, 1).start()
        raw_copy(0, 0).wait()
        start_gathers(0)

        def half_step(g, buf):
            nxt = 1 - buf
            wait_gathers(buf)

            @pl.when(g + 1 < _NCHUNK)
            def _():
                raw_copy(g + 1, nxt).wait()
                start_gathers(nxt)

            @pl.when(g + 2 < _NCHUNK)
            def _():
                raw_copy(g + 2, buf).start()

            process(g, buf)

        def outer(i, carry):
            g = i * 2
            half_step(g, 0)
            half_step(g + 1, 1)
            return carry

        lax.fori_loop(0, _NCHUNK // 2, outer, 0)

    return category_kernel


_kernel_call = _make_kernel()


def kernel(indices, table):
    idx_flat = indices.astype(jnp.int32).reshape(_B * _L)
    p2 = _relayout(table)
    return _kernel_call(idx_flat, table, p2)
